# baseline (speedup 1.0000x reference)
import jax
import jax.numpy as jnp
from jax import lax
from jax.experimental import pallas as pl
from jax.experimental.pallas import tpu as pltpu
from jax.experimental.pallas import tpu_sc as plsc

N = 10000
E = 320000
D = 128
H = 128

NC = 2
NS = 16
EC = E // NS
EW = EC // NC
CH = 2000
KG = 80
SUBS = CH // KG
NPAD = 10240
DR = NPAD // 128
ZU = 80
NU = N // ZU

_F32 = jnp.float32


def _tc_proj(x_ref, w1_ref, wf12_ref, bf12_ref, sf_ref, f12_ref):
    sf = jnp.dot(x_ref[...], w1_ref[...], preferred_element_type=_F32)
    sf_ref[...] = sf
    f12_ref[...] = (
        jnp.dot(sf, wf12_ref[...], preferred_element_type=_F32) + bf12_ref[...]
    )


def _tc_combine(p_ref, w_ref, m_ref, b_ref, o_ref):
    m = m_ref[...]
    s = jnp.exp(m - jnp.max(m))
    s3 = s[:, :1][:, :, None]
    u = jnp.sum(p_ref[...] * s3, axis=0)
    w = jnp.sum(w_ref[...] * s3, axis=0) + 1e-9
    v = u / w + b_ref[...]
    o_ref[...] = jnp.where(v > 0, v, jnp.exp(v) - 1.0)


def _sc_body(f1_hbm, f2_hbm, row_hbm, col_hbm, sf_hbm,
             out_hbm, outw_hbm, outm_hbm,
             tab0, tab1, tab2, ex3_v, rb_v, cb_v, ridx0, ridx1, ridx2,
             idx_v, cbuf, mx_v, mxall_v,
             vals_sh, dfull_sh, mxstage_sh,
             gsem0, gsem1, gsem2, ssem0, ssem1, ssem2):
    c = lax.axis_index("c")
    t = lax.axis_index("s")

    zer16f = jnp.zeros((16,), _F32)

    def zg(i, _):
        for j in range(H // 16):
            tab0[i, pl.ds(j * 16, 16)] = zer16f
        return 0

    lax.fori_loop(0, KG, zg, 0)

    def zv(i, _):
        u = t + i * NS
        @pl.when(u < NU)
        def _():
            pltpu.sync_copy(tab0, vals_sh.at[pl.ds(u * ZU, ZU)])
        return 0

    lax.fori_loop(0, (NU + NS - 1) // NS, zv, 0)

    @pl.when(t < DR // 8)
    def _():
        pltpu.sync_copy(tab0.at[pl.ds(0, 8)], dfull_sh.at[pl.ds(t * 8, 8)])

    pltpu.sync_copy(f1_hbm, tab0)
    pltpu.sync_copy(f2_hbm, tab1)

    def p1c(ch, mx):
        base = t * EC + c * EW + ch * CH
        pltpu.sync_copy(row_hbm.at[pl.ds(base, CH)], rb_v)
        pltpu.sync_copy(col_hbm.at[pl.ds(base, CH)], cb_v)

        def p1(g, mx):
            sl = pl.ds(g * 16, 16)
            rv = rb_v[sl]
            cv = cb_v[sl]
            f1g = plsc.load_gather(tab0, [rv >> 7, rv & 127])
            f2g = plsc.load_gather(tab1, [cv >> 7, cv & 127])
            e = f1g + f2g
            e = jnp.where(e >= 0.0, e, 0.2 * e)
            ex3_v[pl.ds(ch * CH + g * 16, 16)] = e
            return jnp.maximum(mx, e)

        return lax.fori_loop(0, CH // 16, p1, mx)

    mx = lax.fori_loop(0, EW // CH, p1c,
                       jnp.full((16,), -3.0e38, _F32))

    mx_v[...] = mx
    pltpu.sync_copy(mx_v, mxstage_sh.at[pl.ds(t * 16, 16)])
    plsc.subcore_barrier()
    pltpu.sync_copy(mxstage_sh, mxall_v)
    for i in range(NS):
        mx = jnp.maximum(mx, mxall_v[pl.ds(i * 16, 16)])
    gmax = jnp.max(mx)
    mx_v[...] = jnp.full((16,), 0.0, _F32) + gmax

    @pl.when(t == 0)
    def _():
        pltpu.sync_copy(mx_v, outm_hbm.at[c])

    def zden(i, _):
        for v in range(128 // 16):
            tab2[i, pl.ds(v * 16, 16)] = zer16f
        return 0

    lax.fori_loop(0, DR, zden, 0)

    def p2c(ch, _):
        base = t * EC + c * EW + ch * CH
        pltpu.sync_copy(row_hbm.at[pl.ds(base, CH)], rb_v)

        def p2(g, _):
            sl = pl.ds(ch * CH + g * 16, 16)
            ex = jnp.exp(ex3_v[sl] - gmax)
            ex3_v[sl] = ex
            rv = rb_v[pl.ds(g * 16, 16)]
            plsc.addupdate_scatter(tab2, [rv >> 7, rv & 127], ex)
            return 0

        return lax.fori_loop(0, CH // 16, p2, 0)

    lax.fori_loop(0, EW // CH, p2c, 0)

    iota16 = lax.iota(jnp.int32, 16)
    for i in range(DR // 16):
        idx_v[pl.ds(i * 16, 16)] = iota16 + jnp.int32(i * 16)
    plsc.subcore_barrier()
    pltpu.sync_copy(tab2, dfull_sh.at[idx_v], add=True)
    plsc.subcore_barrier()

    @pl.when(t == 0)
    def _():
        pltpu.sync_copy(dfull_sh, outw_hbm.at[c])

    def gstart(s, buf, sem):
        pltpu.async_copy(sf_hbm.at[cb_v.at[pl.ds(s * KG, KG)]], buf, sem)

    def gwait(s, buf, sem):
        pltpu.make_async_copy(
            sf_hbm.at[cb_v.at[pl.ds(s * KG, KG)]], buf, sem).wait()

    def sstart(buf, ridx, sem):
        pltpu.async_copy(buf, vals_sh.at[ridx], sem, add=True)

    def swait(buf, ridx, sem):
        pltpu.make_async_copy(buf, vals_sh.at[ridx], sem).wait()

    def scale(buf, ridx, s, exoff):
        for v in range(KG // 16):
            rv = rb_v[pl.ds(s * KG + v * 16, 16)]
            ridx[pl.ds(v * 16, 16)] = rv
            coef = ex3_v[pl.ds(exoff + v * 16, 16)]
            cbuf[pl.ds(0, 16)] = coef

            def quad(m, _):
                cq = cbuf[pl.ds(4 * m, 16)]
                for q in range(4):
                    ck = cq[q]
                    r = v * 16 + 4 * m + q
                    for j in range(H // 16):
                        sj = pl.ds(j * 16, 16)
                        buf[r, sj] = buf[r, sj] * ck
                return 0

            lax.fori_loop(0, 4, quad, 0)

    bufs = (tab0, tab1, tab2)
    rixs = (ridx0, ridx1, ridx2)
    gsems = (gsem0, gsem1, gsem2)
    ssems = (ssem0, ssem1, ssem2)

    def p3c(ch, _):
        base = t * EC + c * EW + ch * CH
        pltpu.sync_copy(row_hbm.at[pl.ds(base, CH)], rb_v)
        pltpu.sync_copy(col_hbm.at[pl.ds(base, CH)], cb_v)
        exb = ch * CH
        gstart(0, tab0, gsem0)
        gstart(1, tab1, gsem1)
        gstart(2, tab2, gsem2)

        def step(b, s):
            gwait(s, bufs[b], gsems[b])
            scale(bufs[b], rixs[b], s, exb + s * KG)
            sstart(bufs[b], rixs[b], ssems[b])
            b1 = (b + 2) % 3

            @pl.when(s >= 1)
            def _():
                swait(bufs[b1], rixs[b1], ssems[b1])

                @pl.when(s + 2 < SUBS)
                def _():
                    gstart(s + 2, bufs[b1], gsems[b1])

        def tri(i, _):
            s0 = 3 * i
            step(0, s0)
            step(1, s0 + 1)
            step(2, s0 + 2)
            return 0

        lax.fori_loop(0, SUBS // 3, tri, 0)

        s_last = SUBS - 1
        step(0, jnp.int32(s_last))
        swait(bufs[0], rixs[0], ssems[0])
        return 0

    lax.fori_loop(0, EW // CH, p3c, 0)
    plsc.subcore_barrier()

    def dump(i, _):
        u = t + i * NS
        @pl.when(u < NU)
        def _():
            pltpu.sync_copy(vals_sh.at[pl.ds(u * ZU, ZU)], out_hbm.at[c, u])
        return 0

    lax.fori_loop(0, (NU + NS - 1) // NS, dump, 0)


def _build_sc():
    mesh = plsc.VectorSubcoreMesh(
        core_axis_name="c", subcore_axis_name="s", num_cores=NC,
        num_subcores=NS)
    return pl.kernel(
        _sc_body,
        out_type=(
            jax.ShapeDtypeStruct((NC, NU, ZU, H), _F32),
            jax.ShapeDtypeStruct((NC, DR, 128), _F32),
            jax.ShapeDtypeStruct((NC, 16), _F32),
        ),
        mesh=mesh,
        compiler_params=pltpu.CompilerParams(needs_layout_passes=False),
        scratch_types=[
            pltpu.VMEM((DR, 128), _F32),
            pltpu.VMEM((DR, 128), _F32),
            pltpu.VMEM((DR, 128), _F32),
            pltpu.VMEM((EW,), _F32),
            pltpu.VMEM((CH,), jnp.int32),
            pltpu.VMEM((CH,), jnp.int32),
            pltpu.VMEM((KG,), jnp.int32),
            pltpu.VMEM((KG,), jnp.int32),
            pltpu.VMEM((KG,), jnp.int32),
            pltpu.VMEM((DR,), jnp.int32),
            pltpu.VMEM((32,), _F32),
            pltpu.VMEM((16,), _F32),
            pltpu.VMEM((NS * 16,), _F32),
            pltpu.VMEM_SHARED((N, H), _F32),
            pltpu.VMEM_SHARED((DR, 128), _F32),
            pltpu.VMEM_SHARED((NS * 16,), _F32),
            pltpu.SemaphoreType.DMA,
            pltpu.SemaphoreType.DMA,
            pltpu.SemaphoreType.DMA,
            pltpu.SemaphoreType.DMA,
            pltpu.SemaphoreType.DMA,
            pltpu.SemaphoreType.DMA,
        ],
    )


def kernel(seq, edge_index, training, msk_in, W1, wf1, bf1, wf2, bf2,
           bias_zero):
    x = seq[0]
    wf12 = jnp.concatenate([wf1, wf2], axis=1)
    bf12 = jnp.stack([bf1[0], bf2[0]])[None, :]

    grid = N // 1000
    sf, f12 = pl.pallas_call(
        _tc_proj,
        grid=(grid,),
        in_specs=[
            pl.BlockSpec((1000, D), lambda i: (i, 0)),
            pl.BlockSpec((D, H), lambda i: (0, 0)),
            pl.BlockSpec((H, 2), lambda i: (0, 0)),
            pl.BlockSpec((1, 2), lambda i: (0, 0)),
        ],
        out_specs=[
            pl.BlockSpec((1000, H), lambda i: (i, 0)),
            pl.BlockSpec((1000, 2), lambda i: (i, 0)),
        ],
        out_shape=[
            jax.ShapeDtypeStruct((N, H), _F32),
            jax.ShapeDtypeStruct((N, 2), _F32),
        ],
    )(x, W1, wf12, bf12)

    pad = NPAD - N
    f1 = jnp.pad(f12[:, 0], (0, pad)).reshape(DR, 128)
    f2 = jnp.pad(f12[:, 1], (0, pad)).reshape(DR, 128)
    row = edge_index[0].astype(jnp.int32)
    col = edge_index[1].astype(jnp.int32)

    partials, wpart, mpart = _build_sc()(f1, f2, row, col, sf)
    partials = partials.reshape(NC, N, H)
    wpart = wpart.reshape(NC, NPAD)[:, :N, None]

    out = pl.pallas_call(
        _tc_combine,
        grid=(grid,),
        in_specs=[
            pl.BlockSpec((NC, 1000, H), lambda i: (0, i, 0)),
            pl.BlockSpec((NC, 1000, 1), lambda i: (0, i, 0)),
            pl.BlockSpec((NC, 16), lambda i: (0, 0)),
            pl.BlockSpec((1, H), lambda i: (0, 0)),
        ],
        out_specs=pl.BlockSpec((1000, H), lambda i: (i, 0)),
        out_shape=jax.ShapeDtypeStruct((N, H), _F32),
    )(partials, wpart, mpart, bias_zero[None, :])

    return out[None]


# speedup vs baseline: 1.0056x; 1.0056x over previous
import jax
import jax.numpy as jnp
from jax import lax
from jax.experimental import pallas as pl
from jax.experimental.pallas import tpu as pltpu
from jax.experimental.pallas import tpu_sc as plsc

N = 10000
E = 320000
D = 128
H = 128

NC = 2
NS = 16
EC = E // NS
EW = EC // NC
CH = 2000
KG = 80
SUBS = CH // KG
NPAD = 10240
DR = NPAD // 128
ZU = 80
NU = N // ZU

_F32 = jnp.float32


def _tc_proj(x_ref, w1_ref, wf12_ref, bf12_ref, sf_ref, f12_ref):
    sf = jnp.dot(x_ref[...], w1_ref[...], preferred_element_type=_F32)
    sf_ref[...] = sf
    f12_ref[...] = (
        jnp.dot(sf, wf12_ref[...], preferred_element_type=_F32) + bf12_ref[...]
    )


def _tc_combine(p_ref, w_ref, m_ref, b_ref, o_ref):
    m = m_ref[...]
    s = jnp.exp(m - jnp.max(m))
    s3 = s[:, :1][:, :, None]
    u = jnp.sum(p_ref[...] * s3, axis=0)
    w = jnp.sum(w_ref[...] * s3, axis=0) + 1e-9
    v = u / w + b_ref[...]
    o_ref[...] = jnp.where(v > 0, v, jnp.exp(v) - 1.0)


def _sc_body(f1_hbm, f2_hbm, row_hbm, col_hbm, sf_hbm,
             out_hbm, outw_hbm, outm_hbm,
             tab0, tab1, tab2, ex3_v, rb_v, cb_v, ridx0, ridx1, ridx2,
             idx_v, cbuf, mx_v, mxall_v,
             vals_sh, dfull_sh, mxstage_sh,
             gsem0, gsem1, gsem2, ssem0, ssem1, ssem2):
    c = lax.axis_index("c")
    t = lax.axis_index("s")

    zer16f = jnp.zeros((16,), _F32)

    def zg(i, _):
        for j in range(H // 16):
            tab0[i, pl.ds(j * 16, 16)] = zer16f
        return 0

    lax.fori_loop(0, KG, zg, 0)

    def zv(i, _):
        u = t + i * NS
        @pl.when(u < NU)
        def _():
            pltpu.async_copy(tab0, vals_sh.at[pl.ds(u * ZU, ZU)], ssem0)
        return 0

    lax.fori_loop(0, (NU + NS - 1) // NS, zv, 0)

    @pl.when(t < DR // 8)
    def _():
        pltpu.sync_copy(tab0.at[pl.ds(0, 8)], dfull_sh.at[pl.ds(t * 8, 8)])

    pltpu.sync_copy(f2_hbm, tab1)

    def zvw(i, _):
        u = t + i * NS
        @pl.when(u < NU)
        def _():
            pltpu.make_async_copy(
                tab0, vals_sh.at[pl.ds(u * ZU, ZU)], ssem0).wait()
        return 0

    lax.fori_loop(0, (NU + NS - 1) // NS, zvw, 0)
    pltpu.sync_copy(f1_hbm, tab0)

    def p1c(ch, mx):
        base = t * EC + c * EW + ch * CH
        pltpu.sync_copy(row_hbm.at[pl.ds(base, CH)], rb_v)
        pltpu.sync_copy(col_hbm.at[pl.ds(base, CH)], cb_v)

        def p1(g, mx):
            sl = pl.ds(g * 16, 16)
            rv = rb_v[sl]
            cv = cb_v[sl]
            f1g = plsc.load_gather(tab0, [rv >> 7, rv & 127])
            f2g = plsc.load_gather(tab1, [cv >> 7, cv & 127])
            e = f1g + f2g
            e = jnp.where(e >= 0.0, e, 0.2 * e)
            ex3_v[pl.ds(ch * CH + g * 16, 16)] = e
            return jnp.maximum(mx, e)

        return lax.fori_loop(0, CH // 16, p1, mx)

    mx = lax.fori_loop(0, EW // CH, p1c,
                       jnp.full((16,), -3.0e38, _F32))

    mx_v[...] = mx
    pltpu.sync_copy(mx_v, mxstage_sh.at[pl.ds(t * 16, 16)])
    plsc.subcore_barrier()
    pltpu.sync_copy(mxstage_sh, mxall_v)
    for i in range(NS):
        mx = jnp.maximum(mx, mxall_v[pl.ds(i * 16, 16)])
    gmax = jnp.max(mx)
    mx_v[...] = jnp.full((16,), 0.0, _F32) + gmax

    @pl.when(t == 0)
    def _():
        pltpu.sync_copy(mx_v, outm_hbm.at[c])

    def zden(i, _):
        for v in range(128 // 16):
            tab2[i, pl.ds(v * 16, 16)] = zer16f
        return 0

    lax.fori_loop(0, DR, zden, 0)

    def p2c(ch, _):
        base = t * EC + c * EW + ch * CH
        pltpu.sync_copy(row_hbm.at[pl.ds(base, CH)], rb_v)

        def p2(g, _):
            sl = pl.ds(ch * CH + g * 16, 16)
            ex = jnp.exp(ex3_v[sl] - gmax)
            ex3_v[sl] = ex
            rv = rb_v[pl.ds(g * 16, 16)]
            plsc.addupdate_scatter(tab2, [rv >> 7, rv & 127], ex)
            return 0

        return lax.fori_loop(0, CH // 16, p2, 0)

    lax.fori_loop(0, EW // CH, p2c, 0)

    iota16 = lax.iota(jnp.int32, 16)
    for i in range(DR // 16):
        idx_v[pl.ds(i * 16, 16)] = iota16 + jnp.int32(i * 16)
    plsc.subcore_barrier()
    pltpu.sync_copy(tab2, dfull_sh.at[idx_v], add=True)
    plsc.subcore_barrier()

    @pl.when(t == 0)
    def _():
        pltpu.sync_copy(dfull_sh, outw_hbm.at[c])

    def gstart(s, buf, sem):
        pltpu.async_copy(sf_hbm.at[cb_v.at[pl.ds(s * KG, KG)]], buf, sem)

    def gwait(s, buf, sem):
        pltpu.make_async_copy(
            sf_hbm.at[cb_v.at[pl.ds(s * KG, KG)]], buf, sem).wait()

    def sstart(buf, ridx, sem):
        pltpu.async_copy(buf, vals_sh.at[ridx], sem, add=True)

    def swait(buf, ridx, sem):
        pltpu.make_async_copy(buf, vals_sh.at[ridx], sem).wait()

    def scale(buf, ridx, s, exoff):
        for v in range(KG // 16):
            rv = rb_v[pl.ds(s * KG + v * 16, 16)]
            ridx[pl.ds(v * 16, 16)] = rv
            coef = ex3_v[pl.ds(exoff + v * 16, 16)]
            cbuf[pl.ds(0, 16)] = coef

            def quad(m, _):
                cq = cbuf[pl.ds(4 * m, 16)]
                for q in range(4):
                    ck = cq[q]
                    r = v * 16 + 4 * m + q
                    for j in range(H // 16):
                        sj = pl.ds(j * 16, 16)
                        buf[r, sj] = buf[r, sj] * ck
                return 0

            lax.fori_loop(0, 4, quad, 0)

    bufs = (tab0, tab1, tab2)
    rixs = (ridx0, ridx1, ridx2)
    gsems = (gsem0, gsem1, gsem2)
    ssems = (ssem0, ssem1, ssem2)

    def p3c(ch, _):
        base = t * EC + c * EW + ch * CH
        pltpu.sync_copy(row_hbm.at[pl.ds(base, CH)], rb_v)
        pltpu.sync_copy(col_hbm.at[pl.ds(base, CH)], cb_v)
        exb = ch * CH
        gstart(0, tab0, gsem0)
        gstart(1, tab1, gsem1)
        gstart(2, tab2, gsem2)

        def step(b, s):
            gwait(s, bufs[b], gsems[b])
            scale(bufs[b], rixs[b], s, exb + s * KG)
            sstart(bufs[b], rixs[b], ssems[b])
            b1 = (b + 2) % 3

            @pl.when(s >= 1)
            def _():
                swait(bufs[b1], rixs[b1], ssems[b1])

                @pl.when(s + 2 < SUBS)
                def _():
                    gstart(s + 2, bufs[b1], gsems[b1])

        def tri(i, _):
            s0 = 3 * i
            step(0, s0)
            step(1, s0 + 1)
            step(2, s0 + 2)
            return 0

        lax.fori_loop(0, SUBS // 3, tri, 0)

        s_last = SUBS - 1
        step(0, jnp.int32(s_last))
        swait(bufs[0], rixs[0], ssems[0])
        return 0

    lax.fori_loop(0, EW // CH, p3c, 0)
    plsc.subcore_barrier()

    def dump(i, _):
        u = t + i * NS
        @pl.when(u < NU)
        def _():
            pltpu.async_copy(vals_sh.at[pl.ds(u * ZU, ZU)], out_hbm.at[c, u],
                             gsem0)
        return 0

    lax.fori_loop(0, (NU + NS - 1) // NS, dump, 0)

    def dumpw(i, _):
        u = t + i * NS
        @pl.when(u < NU)
        def _():
            pltpu.make_async_copy(
                vals_sh.at[pl.ds(u * ZU, ZU)], out_hbm.at[c, u], gsem0).wait()
        return 0

    lax.fori_loop(0, (NU + NS - 1) // NS, dumpw, 0)


def _build_sc():
    mesh = plsc.VectorSubcoreMesh(
        core_axis_name="c", subcore_axis_name="s", num_cores=NC,
        num_subcores=NS)
    return pl.kernel(
        _sc_body,
        out_type=(
            jax.ShapeDtypeStruct((NC, NU, ZU, H), _F32),
            jax.ShapeDtypeStruct((NC, DR, 128), _F32),
            jax.ShapeDtypeStruct((NC, 16), _F32),
        ),
        mesh=mesh,
        compiler_params=pltpu.CompilerParams(needs_layout_passes=False),
        scratch_types=[
            pltpu.VMEM((DR, 128), _F32),
            pltpu.VMEM((DR, 128), _F32),
            pltpu.VMEM((DR, 128), _F32),
            pltpu.VMEM((EW,), _F32),
            pltpu.VMEM((CH,), jnp.int32),
            pltpu.VMEM((CH,), jnp.int32),
            pltpu.VMEM((KG,), jnp.int32),
            pltpu.VMEM((KG,), jnp.int32),
            pltpu.VMEM((KG,), jnp.int32),
            pltpu.VMEM((DR,), jnp.int32),
            pltpu.VMEM((32,), _F32),
            pltpu.VMEM((16,), _F32),
            pltpu.VMEM((NS * 16,), _F32),
            pltpu.VMEM_SHARED((N, H), _F32),
            pltpu.VMEM_SHARED((DR, 128), _F32),
            pltpu.VMEM_SHARED((NS * 16,), _F32),
            pltpu.SemaphoreType.DMA,
            pltpu.SemaphoreType.DMA,
            pltpu.SemaphoreType.DMA,
            pltpu.SemaphoreType.DMA,
            pltpu.SemaphoreType.DMA,
            pltpu.SemaphoreType.DMA,
        ],
    )


def kernel(seq, edge_index, training, msk_in, W1, wf1, bf1, wf2, bf2,
           bias_zero):
    x = seq[0]
    wf12 = jnp.concatenate([wf1, wf2], axis=1)
    bf12 = jnp.stack([bf1[0], bf2[0]])[None, :]

    grid = N // 1000
    sf, f12 = pl.pallas_call(
        _tc_proj,
        grid=(grid,),
        in_specs=[
            pl.BlockSpec((1000, D), lambda i: (i, 0)),
            pl.BlockSpec((D, H), lambda i: (0, 0)),
            pl.BlockSpec((H, 2), lambda i: (0, 0)),
            pl.BlockSpec((1, 2), lambda i: (0, 0)),
        ],
        out_specs=[
            pl.BlockSpec((1000, H), lambda i: (i, 0)),
            pl.BlockSpec((1000, 2), lambda i: (i, 0)),
        ],
        out_shape=[
            jax.ShapeDtypeStruct((N, H), _F32),
            jax.ShapeDtypeStruct((N, 2), _F32),
        ],
    )(x, W1, wf12, bf12)

    pad = NPAD - N
    f1 = jnp.pad(f12[:, 0], (0, pad)).reshape(DR, 128)
    f2 = jnp.pad(f12[:, 1], (0, pad)).reshape(DR, 128)
    row = edge_index[0].astype(jnp.int32)
    col = edge_index[1].astype(jnp.int32)

    partials, wpart, mpart = _build_sc()(f1, f2, row, col, sf)
    partials = partials.reshape(NC, N, H)
    wpart = wpart.reshape(NC, NPAD)[:, :N, None]

    out = pl.pallas_call(
        _tc_combine,
        grid=(grid,),
        in_specs=[
            pl.BlockSpec((NC, 1000, H), lambda i: (0, i, 0)),
            pl.BlockSpec((NC, 1000, 1), lambda i: (0, i, 0)),
            pl.BlockSpec((NC, 16), lambda i: (0, 0)),
            pl.BlockSpec((1, H), lambda i: (0, 0)),
        ],
        out_specs=pl.BlockSpec((1000, H), lambda i: (i, 0)),
        out_shape=jax.ShapeDtypeStruct((N, H), _F32),
    )(partials, wpart, mpart, bias_zero[None, :])

    return out[None]
